# baseline (speedup 1.0000x reference)
import jax
import jax.numpy as jnp
from jax import lax
from jax.experimental import pallas as pl

B = 4096
N = 32
VD = 256
MD = 6
WS = 256
MT = 768
HM = 512
GH = 384
WRITE_TH = 0.55
CONTR_TH = 0.6
TEMP = 0.25
DECAY = 0.995

R = 128


def _fused(ws_ref, slow_ref, vals_ref, conf_ref, exp_ref, contr_ref,
           alive_ref, md_ref,
           w1a_ref, w1b_ref, b1_ref, w2_ref, b2_ref,
           wg1a_ref, wg1b_ref, wg1c_ref, wg1d_ref, bg1_ref, wg2_ref, bg2_ref,
           wc1a_ref, wc1b_ref, wc1c_ref, wc1d_ref, bc1_ref, wc2_ref, bc2_ref,
           cand_o, wp_o, cp_o, wm_o, cm_o,
           vals_o, conf_o, exp_o, contr_o, alive_o, md_o):
    ws = ws_ref[...]
    slow = slow_ref[...]

    h = ws @ w1a_ref[...] + slow @ w1b_ref[...] + b1_ref[...]
    h = jax.nn.gelu(h)
    cand = h @ w2_ref[...] + b2_ref[...]

    conf = conf_ref[...]
    alive = alive_ref[...]
    w = alive * conf
    denom = jnp.maximum(jnp.sum(w, axis=1, keepdims=True), 1.0)
    md = md_ref[...]
    msum = jnp.sum(md * w[:, :, None], axis=1) / denom
    msum8 = jnp.concatenate(
        [msum, jnp.zeros((msum.shape[0], 2), msum.dtype)], axis=1)

    gpre = (ws @ wg1a_ref[...] + slow @ wg1b_ref[...]
            + cand @ wg1c_ref[...] + msum8 @ wg1d_ref[...] + bg1_ref[...])
    g = jax.nn.gelu(gpre)
    wp = jax.nn.sigmoid(
        jnp.sum(g * wg2_ref[...], axis=1, keepdims=True) + bg2_ref[...])
    cpre = (ws @ wc1a_ref[...] + slow @ wc1b_ref[...]
            + cand @ wc1c_ref[...] + msum8 @ wc1d_ref[...] + bc1_ref[...])
    c = jax.nn.gelu(cpre)
    cp = jax.nn.sigmoid(
        jnp.sum(c * wc2_ref[...], axis=1, keepdims=True) + bc2_ref[...])

    wmask = jax.nn.sigmoid((wp - WRITE_TH) / TEMP)
    cmask = jax.nn.sigmoid((cp - CONTR_TH) / TEMP)

    expv = exp_ref[...]
    contr = contr_ref[...]
    inactive = (alive < 0.5)
    has_in = jnp.max(inactive.astype(jnp.float32), axis=1, keepdims=True) > 0.5
    iota = lax.broadcasted_iota(jnp.int32, inactive.shape, 1)
    in_idx = jnp.min(jnp.where(inactive, iota, N), axis=1, keepdims=True)
    util = conf * expv * (1.0 - contr)
    umin = jnp.min(util, axis=1, keepdims=True)
    rep_idx = jnp.min(jnp.where(util == umin, iota, N), axis=1, keepdims=True)
    slot = jnp.where(has_in, in_idx, rep_idx)

    hot = (iota == slot).astype(jnp.float32)
    wslot = hot * wmask
    keep = jnp.clip(1.0 - wslot, 0.0, None)

    vals = vals_ref[...]
    vals_o[...] = keep[:, :, None] * vals + wslot[:, :, None] * cand[:, None, :]
    conf_new = keep * (conf * DECAY) + wslot * wmask
    exp_new = keep * (expv * DECAY) + wslot
    contr_new = keep * contr + wslot * cmask
    alive_new = jnp.clip(keep * alive + wslot, 0.0, 1.0)
    conf_o[...] = conf_new
    exp_o[...] = exp_new
    contr_o[...] = contr_new
    alive_o[...] = alive_new
    mdk = keep[:, :, None] * md
    md_o[...] = jnp.concatenate(
        [conf_new[:, :, None], exp_new[:, :, None], contr_new[:, :, None],
         alive_new[:, :, None], mdk[:, :, 4:6]], axis=2)

    cand_o[...] = cand
    wp_o[...] = wp
    cp_o[...] = cp
    wm_o[...] = wmask
    cm_o[...] = cmask


def kernel(workspace, slow_summary_token, state_values, state_confidence,
           state_expiry, state_contradiction, state_alive, state_metadata,
           W1, b1, W2, b2, Wg1, bg1, Wg2, bg2, Wc1, bc1, Wc2, bc2):
    f32 = jnp.float32
    conf2 = state_confidence[..., 0]
    exp2 = state_expiry[..., 0]
    contr2 = state_contradiction[..., 0]
    alive2 = state_alive[..., 0]

    pad2 = jnp.zeros((2, GH), f32)
    w1a, w1b = W1[:WS], W1[WS:]
    wg1a, wg1b, wg1c = Wg1[:WS], Wg1[WS:WS + MT], Wg1[WS + MT:WS + MT + VD]
    wg1d = jnp.concatenate([Wg1[WS + MT + VD:], pad2], axis=0)
    wc1a, wc1b, wc1c = Wc1[:WS], Wc1[WS:WS + MT], Wc1[WS + MT:WS + MT + VD]
    wc1d = jnp.concatenate([Wc1[WS + MT + VD:], pad2], axis=0)

    grid = (B // R,)

    def row(shape):
        return pl.BlockSpec((R,) + shape, lambda i: (i,) + (0,) * len(shape))

    def full(shape):
        return pl.BlockSpec(shape, lambda i: (0,) * len(shape))

    in_specs = [
        row((WS,)), row((MT,)), row((N, VD)),
        row((N,)), row((N,)), row((N,)), row((N,)), row((N, MD)),
        full((WS, HM)), full((MT, HM)), full((1, HM)),
        full((HM, VD)), full((1, VD)),
        full((WS, GH)), full((MT, GH)), full((VD, GH)), full((8, GH)),
        full((1, GH)), full((1, GH)), full((1, 1)),
        full((WS, GH)), full((MT, GH)), full((VD, GH)), full((8, GH)),
        full((1, GH)), full((1, GH)), full((1, 1)),
    ]
    out_specs = [
        row((VD,)), row((1,)), row((1,)), row((1,)), row((1,)),
        row((N, VD)), row((N,)), row((N,)), row((N,)), row((N,)),
        row((N, MD)),
    ]
    out_shapes = [
        jax.ShapeDtypeStruct((B, VD), f32),
        jax.ShapeDtypeStruct((B, 1), f32),
        jax.ShapeDtypeStruct((B, 1), f32),
        jax.ShapeDtypeStruct((B, 1), f32),
        jax.ShapeDtypeStruct((B, 1), f32),
        jax.ShapeDtypeStruct((B, N, VD), f32),
        jax.ShapeDtypeStruct((B, N), f32),
        jax.ShapeDtypeStruct((B, N), f32),
        jax.ShapeDtypeStruct((B, N), f32),
        jax.ShapeDtypeStruct((B, N), f32),
        jax.ShapeDtypeStruct((B, N, MD), f32),
    ]

    outs = pl.pallas_call(
        _fused,
        grid=grid,
        in_specs=in_specs,
        out_specs=out_specs,
        out_shape=out_shapes,
    )(workspace, slow_summary_token, state_values,
      conf2, exp2, contr2, alive2, state_metadata,
      w1a, w1b, b1[None, :], W2, b2[None, :],
      wg1a, wg1b, wg1c, wg1d, bg1[None, :], Wg2.reshape(1, GH),
      bg2.reshape(1, 1),
      wc1a, wc1b, wc1c, wc1d, bc1[None, :], Wc2.reshape(1, GH),
      bc2.reshape(1, 1))

    (cand, wp, cp, wm, cm, vals, conf_n, exp_n, contr_n, alive_n, md_n) = outs
    return (cand, wp, cp, wm, cm, vals,
            conf_n[..., None], exp_n[..., None], contr_n[..., None],
            alive_n[..., None], md_n)


# speedup vs baseline: 1.6377x; 1.6377x over previous
import jax
import jax.numpy as jnp
from jax import lax
from jax.experimental import pallas as pl

B = 4096
N = 32
VD = 256
MD = 6
WS = 256
MT = 768
HM = 512
GH = 384
WRITE_TH = 0.55
CONTR_TH = 0.6
TEMP = 0.25
DECAY = 0.995

R = 128


def _fused(ws_ref, slow_ref, vals_ref, conf_ref, exp_ref, contr_ref,
           alive_ref, md0_ref, md1_ref, md2_ref, md3_ref, md4_ref, md5_ref,
           w1a_ref, w1b_ref, b1_ref, w2_ref, b2_ref,
           wg1a_ref, wg1b_ref, wg1c_ref, wg1d_ref, bg1_ref, wg2_ref, bg2_ref,
           wc1a_ref, wc1b_ref, wc1c_ref, wc1d_ref, bc1_ref, wc2_ref, bc2_ref,
           cand_o, wp_o, cp_o, wm_o, cm_o,
           vals_o, conf_o, exp_o, contr_o, alive_o, md4_o, md5_o):
    ws = ws_ref[...]
    slow = slow_ref[...]

    h = ws @ w1a_ref[...] + slow @ w1b_ref[...] + b1_ref[...]
    h = jax.nn.gelu(h)
    cand = h @ w2_ref[...] + b2_ref[...]

    conf = conf_ref[...]
    alive = alive_ref[...]
    w = alive * conf
    denom = jnp.maximum(jnp.sum(w, axis=1, keepdims=True), 1.0)
    md4 = md4_ref[...]
    md5 = md5_ref[...]
    cols = (md0_ref[...], md1_ref[...], md2_ref[...], md3_ref[...], md4, md5)
    msum8 = jnp.concatenate(
        [jnp.sum(c * w, axis=1, keepdims=True) for c in cols]
        + [jnp.zeros((w.shape[0], 2), w.dtype)], axis=1) / denom

    gpre = (ws @ wg1a_ref[...] + slow @ wg1b_ref[...]
            + cand @ wg1c_ref[...] + msum8 @ wg1d_ref[...] + bg1_ref[...])
    g = jax.nn.gelu(gpre)
    wp = jax.nn.sigmoid(
        jnp.sum(g * wg2_ref[...], axis=1, keepdims=True) + bg2_ref[...])
    cpre = (ws @ wc1a_ref[...] + slow @ wc1b_ref[...]
            + cand @ wc1c_ref[...] + msum8 @ wc1d_ref[...] + bc1_ref[...])
    c = jax.nn.gelu(cpre)
    cp = jax.nn.sigmoid(
        jnp.sum(c * wc2_ref[...], axis=1, keepdims=True) + bc2_ref[...])

    wmask = jax.nn.sigmoid((wp - WRITE_TH) / TEMP)
    cmask = jax.nn.sigmoid((cp - CONTR_TH) / TEMP)

    expv = exp_ref[...]
    contr = contr_ref[...]
    inactive = (alive < 0.5)
    has_in = jnp.max(inactive.astype(jnp.float32), axis=1, keepdims=True) > 0.5
    iota = lax.broadcasted_iota(jnp.int32, inactive.shape, 1)
    in_idx = jnp.min(jnp.where(inactive, iota, N), axis=1, keepdims=True)
    util = conf * expv * (1.0 - contr)
    umin = jnp.min(util, axis=1, keepdims=True)
    rep_idx = jnp.min(jnp.where(util == umin, iota, N), axis=1, keepdims=True)
    slot = jnp.where(has_in, in_idx, rep_idx)

    hot = (iota == slot)
    wslot = jnp.where(hot, wmask, 0.0)
    keep = 1.0 - wslot
    conf_o[...] = keep * (conf * DECAY) + wslot * wmask
    exp_o[...] = keep * (expv * DECAY) + wslot
    contr_o[...] = keep * contr + wslot * cmask
    alive_o[...] = jnp.clip(keep * alive + wslot, 0.0, 1.0)
    md4_o[...] = keep * md4
    md5_o[...] = keep * md5

    vals = vals_ref[...]
    iota3 = lax.broadcasted_iota(jnp.int32, (vals.shape[0], N, 1), 1)
    wslot3 = jnp.where(iota3 == slot[:, :, None], wmask[:, :, None], 0.0)
    vals_o[...] = vals + wslot3 * (cand[:, None, :] - vals)

    cand_o[...] = cand
    wp_o[...] = wp
    cp_o[...] = cp
    wm_o[...] = wmask
    cm_o[...] = cmask


def kernel(workspace, slow_summary_token, state_values, state_confidence,
           state_expiry, state_contradiction, state_alive, state_metadata,
           W1, b1, W2, b2, Wg1, bg1, Wg2, bg2, Wc1, bc1, Wc2, bc2):
    f32 = jnp.float32
    conf2 = state_confidence[..., 0]
    exp2 = state_expiry[..., 0]
    contr2 = state_contradiction[..., 0]
    alive2 = state_alive[..., 0]
    mdcols = tuple(state_metadata[..., j] for j in range(MD))

    pad2 = jnp.zeros((2, GH), f32)
    w1a, w1b = W1[:WS], W1[WS:]
    wg1a, wg1b, wg1c = Wg1[:WS], Wg1[WS:WS + MT], Wg1[WS + MT:WS + MT + VD]
    wg1d = jnp.concatenate([Wg1[WS + MT + VD:], pad2], axis=0)
    wc1a, wc1b, wc1c = Wc1[:WS], Wc1[WS:WS + MT], Wc1[WS + MT:WS + MT + VD]
    wc1d = jnp.concatenate([Wc1[WS + MT + VD:], pad2], axis=0)

    grid = (B // R,)

    def row(shape):
        return pl.BlockSpec((R,) + shape, lambda i: (i,) + (0,) * len(shape))

    def full(shape):
        return pl.BlockSpec(shape, lambda i: (0,) * len(shape))

    in_specs = [
        row((WS,)), row((MT,)), row((N, VD)),
        row((N,)), row((N,)), row((N,)), row((N,)),
        row((N,)), row((N,)), row((N,)), row((N,)), row((N,)), row((N,)),
        full((WS, HM)), full((MT, HM)), full((1, HM)),
        full((HM, VD)), full((1, VD)),
        full((WS, GH)), full((MT, GH)), full((VD, GH)), full((8, GH)),
        full((1, GH)), full((1, GH)), full((1, 1)),
        full((WS, GH)), full((MT, GH)), full((VD, GH)), full((8, GH)),
        full((1, GH)), full((1, GH)), full((1, 1)),
    ]
    out_specs = [
        row((VD,)), row((1,)), row((1,)), row((1,)), row((1,)),
        row((N, VD)), row((N,)), row((N,)), row((N,)), row((N,)),
        row((N,)), row((N,)),
    ]
    out_shapes = [
        jax.ShapeDtypeStruct((B, VD), f32),
        jax.ShapeDtypeStruct((B, 1), f32),
        jax.ShapeDtypeStruct((B, 1), f32),
        jax.ShapeDtypeStruct((B, 1), f32),
        jax.ShapeDtypeStruct((B, 1), f32),
        jax.ShapeDtypeStruct((B, N, VD), f32),
        jax.ShapeDtypeStruct((B, N), f32),
        jax.ShapeDtypeStruct((B, N), f32),
        jax.ShapeDtypeStruct((B, N), f32),
        jax.ShapeDtypeStruct((B, N), f32),
        jax.ShapeDtypeStruct((B, N), f32),
        jax.ShapeDtypeStruct((B, N), f32),
    ]

    outs = pl.pallas_call(
        _fused,
        grid=grid,
        in_specs=in_specs,
        out_specs=out_specs,
        out_shape=out_shapes,
    )(workspace, slow_summary_token, state_values,
      conf2, exp2, contr2, alive2, *mdcols,
      w1a, w1b, b1[None, :], W2, b2[None, :],
      wg1a, wg1b, wg1c, wg1d, bg1[None, :], Wg2.reshape(1, GH),
      bg2.reshape(1, 1),
      wc1a, wc1b, wc1c, wc1d, bc1[None, :], Wc2.reshape(1, GH),
      bc2.reshape(1, 1))

    (cand, wp, cp, wm, cm, vals, conf_n, exp_n, contr_n, alive_n,
     md4_n, md5_n) = outs
    metadata = jnp.stack([conf_n, exp_n, contr_n, alive_n, md4_n, md5_n],
                         axis=-1)
    return (cand, wp, cp, wm, cm, vals,
            conf_n[..., None], exp_n[..., None], contr_n[..., None],
            alive_n[..., None], metadata)


# speedup vs baseline: 1.7617x; 1.0757x over previous
import jax
import jax.numpy as jnp
from jax import lax
from jax.experimental import pallas as pl

B = 4096
N = 32
VD = 256
MD = 6
WS = 256
MT = 768
HM = 512
GH = 384
WRITE_TH = 0.55
CONTR_TH = 0.6
TEMP = 0.25
DECAY = 0.995

R = 256


def _fused(ws_ref, slow_ref, vals_ref, conf_ref, exp_ref, contr_ref,
           alive_ref, md0_ref, md1_ref, md2_ref, md3_ref, md4_ref, md5_ref,
           w1_0, w1_1, w1_2, w1_3, b1_ref, w2_ref, b2_ref,
           wg_0, wg_1, wg_2, wg_3, wg_4, wg1d_ref, bg1_ref, wg2_ref, bg2_ref,
           wc_0, wc_1, wc_2, wc_3, wc_4, wc1d_ref, bc1_ref, wc2_ref, bc2_ref,
           cand_o, probs_o, vals_o, conf_o, exp_o, contr_o, alive_o, md_o):
    ws = ws_ref[...]
    slow = slow_ref[...]
    s0 = slow[:, :256]
    s1 = slow[:, 256:512]
    s2 = slow[:, 512:]

    h = (ws @ w1_0[...] + s0 @ w1_1[...] + s1 @ w1_2[...] + s2 @ w1_3[...]
         + b1_ref[...])
    h = jax.nn.gelu(h)
    cand = h @ w2_ref[...] + b2_ref[...]

    conf = conf_ref[...]
    alive = alive_ref[...]
    w = alive * conf
    denom = jnp.maximum(jnp.sum(w, axis=1, keepdims=True), 1.0)
    md4 = md4_ref[...]
    md5 = md5_ref[...]
    cols = (md0_ref[...], md1_ref[...], md2_ref[...], md3_ref[...], md4, md5)
    msum8 = jnp.concatenate(
        [jnp.sum(c * w, axis=1, keepdims=True) for c in cols]
        + [jnp.zeros((w.shape[0], 2), w.dtype)], axis=1) / denom

    gpre = (ws @ wg_0[...] + s0 @ wg_1[...] + s1 @ wg_2[...] + s2 @ wg_3[...]
            + cand @ wg_4[...] + msum8 @ wg1d_ref[...] + bg1_ref[...])
    g = jax.nn.gelu(gpre)
    wp = jax.nn.sigmoid(
        jnp.sum(g * wg2_ref[...], axis=1, keepdims=True) + bg2_ref[...])
    cpre = (ws @ wc_0[...] + s0 @ wc_1[...] + s1 @ wc_2[...] + s2 @ wc_3[...]
            + cand @ wc_4[...] + msum8 @ wc1d_ref[...] + bc1_ref[...])
    c = jax.nn.gelu(cpre)
    cp = jax.nn.sigmoid(
        jnp.sum(c * wc2_ref[...], axis=1, keepdims=True) + bc2_ref[...])

    wmask = jax.nn.sigmoid((wp - WRITE_TH) / TEMP)
    cmask = jax.nn.sigmoid((cp - CONTR_TH) / TEMP)

    expv = exp_ref[...]
    contr = contr_ref[...]
    inactive = (alive < 0.5)
    has_in = jnp.max(inactive.astype(jnp.float32), axis=1, keepdims=True) > 0.5
    iota = lax.broadcasted_iota(jnp.int32, inactive.shape, 1)
    in_idx = jnp.min(jnp.where(inactive, iota, N), axis=1, keepdims=True)
    util = conf * expv * (1.0 - contr)
    umin = jnp.min(util, axis=1, keepdims=True)
    rep_idx = jnp.min(jnp.where(util == umin, iota, N), axis=1, keepdims=True)
    slot = jnp.where(has_in, in_idx, rep_idx)

    hot = (iota == slot)
    wslot = jnp.where(hot, wmask, 0.0)
    keep = 1.0 - wslot
    conf_n = keep * (conf * DECAY) + wslot * wmask
    exp_n = keep * (expv * DECAY) + wslot
    contr_n = keep * contr + wslot * cmask
    alive_n = jnp.clip(keep * alive + wslot, 0.0, 1.0)
    conf_o[...] = conf_n
    exp_o[...] = exp_n
    contr_o[...] = contr_n
    alive_o[...] = alive_n
    md_o[...] = jnp.concatenate(
        [conf_n, exp_n, contr_n, alive_n, keep * md4, keep * md5], axis=1)

    vals = vals_ref[...]
    iota3 = lax.broadcasted_iota(jnp.int32, (vals.shape[0], N, 1), 1)
    wslot3 = jnp.where(iota3 == slot[:, :, None], wmask[:, :, None], 0.0)
    vals_o[...] = vals + wslot3 * (cand[:, None, :] - vals)

    cand_o[...] = cand
    probs_o[...] = jnp.concatenate([wp, cp, wmask, cmask], axis=1)


def kernel(workspace, slow_summary_token, state_values, state_confidence,
           state_expiry, state_contradiction, state_alive, state_metadata,
           W1, b1, W2, b2, Wg1, bg1, Wg2, bg2, Wc1, bc1, Wc2, bc2):
    f32 = jnp.float32
    conf2 = state_confidence[..., 0]
    exp2 = state_expiry[..., 0]
    contr2 = state_contradiction[..., 0]
    alive2 = state_alive[..., 0]
    mdcols = tuple(state_metadata[..., j] for j in range(MD))

    pad2 = jnp.zeros((2, GH), f32)
    wg1d = jnp.concatenate([Wg1[WS + MT + VD:], pad2], axis=0)
    wc1d = jnp.concatenate([Wc1[WS + MT + VD:], pad2], axis=0)

    grid = (B // R,)

    def row(shape):
        return pl.BlockSpec((R,) + shape, lambda i: (i,) + (0,) * len(shape))

    def full(shape):
        return pl.BlockSpec(shape, lambda i: (0,) * len(shape))

    def chunk(shape, j):
        return pl.BlockSpec(shape, lambda i, j=j: (j, 0))

    in_specs = [
        row((WS,)), row((MT,)), row((N, VD)),
        row((N,)), row((N,)), row((N,)), row((N,)),
        row((N,)), row((N,)), row((N,)), row((N,)), row((N,)), row((N,)),
        chunk((WS, HM), 0), chunk((WS, HM), 1), chunk((WS, HM), 2),
        chunk((WS, HM), 3), full((1, HM)),
        full((HM, VD)), full((1, VD)),
        chunk((WS, GH), 0), chunk((WS, GH), 1), chunk((WS, GH), 2),
        chunk((WS, GH), 3), chunk((WS, GH), 4), full((8, GH)),
        full((1, GH)), full((1, GH)), full((1, 1)),
        chunk((WS, GH), 0), chunk((WS, GH), 1), chunk((WS, GH), 2),
        chunk((WS, GH), 3), chunk((WS, GH), 4), full((8, GH)),
        full((1, GH)), full((1, GH)), full((1, 1)),
    ]
    out_specs = [
        row((VD,)), row((4,)),
        row((N, VD)), row((N,)), row((N,)), row((N,)), row((N,)),
        row((N * MD,)),
    ]
    out_shapes = [
        jax.ShapeDtypeStruct((B, VD), f32),
        jax.ShapeDtypeStruct((B, 4), f32),
        jax.ShapeDtypeStruct((B, N, VD), f32),
        jax.ShapeDtypeStruct((B, N), f32),
        jax.ShapeDtypeStruct((B, N), f32),
        jax.ShapeDtypeStruct((B, N), f32),
        jax.ShapeDtypeStruct((B, N), f32),
        jax.ShapeDtypeStruct((B, N * MD), f32),
    ]

    outs = pl.pallas_call(
        _fused,
        grid=grid,
        in_specs=in_specs,
        out_specs=out_specs,
        out_shape=out_shapes,
    )(workspace, slow_summary_token, state_values,
      conf2, exp2, contr2, alive2, *mdcols,
      W1, W1, W1, W1, b1[None, :], W2, b2[None, :],
      Wg1, Wg1, Wg1, Wg1, Wg1, wg1d, bg1[None, :], Wg2.reshape(1, GH),
      bg2.reshape(1, 1),
      Wc1, Wc1, Wc1, Wc1, Wc1, wc1d, bc1[None, :], Wc2.reshape(1, GH),
      bc2.reshape(1, 1))

    (cand, probs, vals, conf_n, exp_n, contr_n, alive_n, md_n) = outs
    metadata = md_n.reshape(B, MD, N).transpose(0, 2, 1)
    return (cand, probs[:, 0:1], probs[:, 1:2], probs[:, 2:3], probs[:, 3:4],
            vals, conf_n[..., None], exp_n[..., None], contr_n[..., None],
            alive_n[..., None], metadata)
